# baseline (speedup 1.0000x reference)
import functools

import jax
import jax.numpy as jnp
from jax import lax
from jax.experimental import pallas as pl
from jax.experimental.pallas import tpu as pltpu
from jax.experimental.pallas import tpu_sc as plsc

N = 32768
E = 8192
D = 256

BLK_N = 512
BLK_E = 1024
N_STEPS = N // BLK_N
E_STEPS = E // BLK_E


def _argmin_body(xb_ref, ebt_ref, xp_ref, ep_ref, code_ref):
    xb = xb_ref[...]
    xp = xp_ref[...]

    def step(j, carry):
        rmin, ridx = carry
        ebt = ebt_ref[:, pl.ds(j * BLK_E, BLK_E)]
        mm = jax.lax.dot_general(
            xb, ebt, (((1,), (0,)), ((), ())),
            preferred_element_type=jnp.float32)
        ep = ep_ref[:, pl.ds(j * BLK_E, BLK_E)]
        d = (xp + ep) - 2.0 * mm
        bm = jnp.min(d, axis=1, keepdims=True)
        iota = jax.lax.broadcasted_iota(jnp.int32, (BLK_N, BLK_E), 1)
        bidx = jnp.min(jnp.where(d == bm, iota, jnp.int32(2**30)),
                       axis=1, keepdims=True) + j * BLK_E
        better = bm < rmin
        rmin = jnp.where(better, bm, rmin)
        ridx = jnp.where(better, bidx, ridx)
        return rmin, ridx

    init = (jnp.full((BLK_N, 1), jnp.inf, jnp.float32),
            jnp.zeros((BLK_N, 1), jnp.int32))
    _, ridx = jax.lax.fori_loop(0, E_STEPS, step, init)
    code_ref[...] = ridx[:, 0]


def _argmin_call(xb, ebt, xp, ep):
    return pl.pallas_call(
        _argmin_body,
        grid=(N_STEPS,),
        in_specs=[
            pl.BlockSpec((BLK_N, D), lambda i: (i, 0)),
            pl.BlockSpec((D, E), lambda i: (0, 0)),
            pl.BlockSpec((BLK_N, 1), lambda i: (i, 0)),
            pl.BlockSpec((1, E), lambda i: (0, 0)),
        ],
        out_specs=pl.BlockSpec((BLK_N,), lambda i: (i,)),
        out_shape=jax.ShapeDtypeStruct((N,), jnp.int32),
    )(xb, ebt, xp, ep)


def _make_gather():
    info = plsc.get_sparse_core_info()
    NC, NS = info.num_cores, info.num_subcores
    NW = NC * NS
    b_per_w = N // NW
    CHUNK = 256
    n_chunks = b_per_w // CHUNK

    mesh = plsc.VectorSubcoreMesh(core_axis_name="c", subcore_axis_name="s")

    @functools.partial(
        pl.kernel, mesh=mesh,
        out_type=jax.ShapeDtypeStruct((N, D), jnp.float32),
        scratch_types=[
            pltpu.VMEM((CHUNK,), jnp.int32),
            pltpu.VMEM((CHUNK, D), jnp.float32),
            pltpu.SemaphoreType.DMA,
        ],
    )
    def gather(table_hbm, idx_hbm, out_hbm, idx_v, rows_v, sem):
        wid = lax.axis_index("s") * NC + lax.axis_index("c")
        base = wid * b_per_w

        def chunk(c, _):
            off = base + c * CHUNK
            pltpu.sync_copy(idx_hbm.at[pl.ds(off, CHUNK)], idx_v)
            pltpu.async_copy(table_hbm.at[idx_v], rows_v, sem).wait()
            pltpu.sync_copy(rows_v, out_hbm.at[pl.ds(off, CHUNK)])
            return 0

        jax.lax.fori_loop(0, n_chunks, chunk, 0)

    return gather


def kernel(inputs, embeddings):
    x = inputs
    e = embeddings
    et = e.T
    xp = jnp.sum(jnp.power(x, 2), axis=-1, keepdims=True)
    ep = jnp.sum(jnp.power(et, 2), axis=0, keepdims=True)
    xb = x.astype(jnp.bfloat16)
    ebt = et.astype(jnp.bfloat16)
    code = _argmin_call(xb, ebt, xp, ep)
    quant = _make_gather()(e, code)
    return (code, quant)


# speedup vs baseline: 1.6188x; 1.6188x over previous
import functools

import jax
import jax.numpy as jnp
from jax import lax
from jax.experimental import pallas as pl
from jax.experimental.pallas import tpu as pltpu
from jax.experimental.pallas import tpu_sc as plsc

N = 32768
E = 8192
D = 256

BLK_N = 256
N_STEPS = N // BLK_N
LANES = 128
N_CHUNKS = E // LANES


def _argmin_body(xb_ref, ebt_ref, xp_ref, ep_ref, code_ref):
    xb = xb_ref[...]
    xp = xp_ref[...]
    mmn = jax.lax.dot_general(
        xb, ebt_ref[...], (((1,), (0,)), ((), ())),
        preferred_element_type=jnp.float32)

    val = jnp.full((BLK_N, LANES), jnp.inf, jnp.float32)
    grp = jnp.zeros((BLK_N, LANES), jnp.float32)
    for g in range(N_CHUNKS):
        sl = slice(g * LANES, (g + 1) * LANES)
        d_g = mmn[:, sl] + (xp + ep_ref[:, sl])
        better = d_g < val
        val = jnp.minimum(val, d_g)
        grp = jnp.where(better, jnp.float32(g), grp)

    lane = jax.lax.broadcasted_iota(jnp.int32, (BLK_N, LANES), 1).astype(jnp.float32)
    gidx = grp * jnp.float32(LANES) + lane
    bm = jnp.min(val, axis=1, keepdims=True)
    bidx = jnp.min(jnp.where(val == bm, gidx, jnp.inf), axis=1)
    code_ref[...] = bidx.astype(jnp.int32)


def _argmin_call(xb, ebt, xp, ep):
    return pl.pallas_call(
        _argmin_body,
        grid=(N_STEPS,),
        in_specs=[
            pl.BlockSpec((BLK_N, D), lambda i: (i, 0)),
            pl.BlockSpec((D, E), lambda i: (0, 0)),
            pl.BlockSpec((BLK_N, 1), lambda i: (i, 0)),
            pl.BlockSpec((1, E), lambda i: (0, 0)),
        ],
        out_specs=pl.BlockSpec((BLK_N,), lambda i: (i,)),
        out_shape=jax.ShapeDtypeStruct((N,), jnp.int32),
    )(xb, ebt, xp, ep)


def _make_gather():
    info = plsc.get_sparse_core_info()
    NC, NS = info.num_cores, info.num_subcores
    NW = NC * NS
    b_per_w = N // NW
    CHUNK = 256
    n_chunks = b_per_w // CHUNK

    mesh = plsc.VectorSubcoreMesh(core_axis_name="c", subcore_axis_name="s")

    @functools.partial(
        pl.kernel, mesh=mesh,
        out_type=jax.ShapeDtypeStruct((N, D), jnp.float32),
        scratch_types=[
            pltpu.VMEM((CHUNK,), jnp.int32),
            pltpu.VMEM((CHUNK, D), jnp.float32),
            pltpu.SemaphoreType.DMA,
        ],
    )
    def gather(table_hbm, idx_hbm, out_hbm, idx_v, rows_v, sem):
        wid = lax.axis_index("s") * NC + lax.axis_index("c")
        base = wid * b_per_w

        def chunk(c, _):
            off = base + c * CHUNK
            pltpu.sync_copy(idx_hbm.at[pl.ds(off, CHUNK)], idx_v)
            pltpu.async_copy(table_hbm.at[idx_v], rows_v, sem).wait()
            pltpu.sync_copy(rows_v, out_hbm.at[pl.ds(off, CHUNK)])
            return 0

        jax.lax.fori_loop(0, n_chunks, chunk, 0)

    return gather


def kernel(inputs, embeddings):
    x = inputs
    e = embeddings
    et = e.T
    xp = jnp.sum(jnp.power(x, 2), axis=-1, keepdims=True)
    ep = jnp.sum(jnp.power(et, 2), axis=0, keepdims=True)
    xb = (x * jnp.float32(-2.0)).astype(jnp.bfloat16)
    ebt = et.astype(jnp.bfloat16)
    code = _argmin_call(xb, ebt, xp, ep)
    quant = _make_gather()(e, code)
    return (code, quant)


# speedup vs baseline: 1.7544x; 1.0838x over previous
import functools

import jax
import jax.numpy as jnp
from jax import lax
from jax.experimental import pallas as pl
from jax.experimental.pallas import tpu as pltpu
from jax.experimental.pallas import tpu_sc as plsc

N = 32768
E = 8192
D = 256

BLK_N = 512
N_STEPS = N // BLK_N
LANES = 128
N_CHUNKS = E // LANES


def _argmin_body(x_ref, ebt_ref, xp_ref, ep_ref, code_ref):
    xb = x_ref[...].astype(jnp.bfloat16)
    xp = xp_ref[...]
    mmn = jax.lax.dot_general(
        xb, ebt_ref[...], (((1,), (0,)), ((), ())),
        preferred_element_type=jnp.float32)

    val = jnp.full((BLK_N, LANES), jnp.inf, jnp.float32)
    grp = jnp.zeros((BLK_N, LANES), jnp.float32)
    for g in range(N_CHUNKS):
        sl = slice(g * LANES, (g + 1) * LANES)
        d_g = mmn[:, sl] + (xp + ep_ref[:, sl])
        better = d_g < val
        val = jnp.minimum(val, d_g)
        grp = jnp.where(better, jnp.float32(g), grp)

    lane = jax.lax.broadcasted_iota(jnp.int32, (BLK_N, LANES), 1).astype(jnp.float32)
    gidx = grp * jnp.float32(LANES) + lane
    bm = jnp.min(val, axis=1, keepdims=True)
    bidx = jnp.min(jnp.where(val == bm, gidx, jnp.inf), axis=1)
    code_ref[...] = bidx.astype(jnp.int32)


def _argmin_call(x, ebt, xp, ep):
    return pl.pallas_call(
        _argmin_body,
        grid=(N_STEPS,),
        in_specs=[
            pl.BlockSpec((BLK_N, D), lambda i: (i, 0)),
            pl.BlockSpec((D, E), lambda i: (0, 0)),
            pl.BlockSpec((BLK_N, 1), lambda i: (i, 0)),
            pl.BlockSpec((1, E), lambda i: (0, 0)),
        ],
        out_specs=pl.BlockSpec((BLK_N,), lambda i: (i,)),
        out_shape=jax.ShapeDtypeStruct((N,), jnp.int32),
    )(x, ebt, xp, ep)


def _make_gather():
    info = plsc.get_sparse_core_info()
    NC, NS = info.num_cores, info.num_subcores
    NW = NC * NS
    b_per_w = N // NW
    CHUNK = 256
    n_chunks = b_per_w // CHUNK

    mesh = plsc.VectorSubcoreMesh(core_axis_name="c", subcore_axis_name="s")

    @functools.partial(
        pl.kernel, mesh=mesh,
        out_type=jax.ShapeDtypeStruct((N, D), jnp.float32),
        scratch_types=[
            pltpu.VMEM((CHUNK,), jnp.int32),
            pltpu.VMEM((CHUNK, D), jnp.float32),
            pltpu.SemaphoreType.DMA,
        ],
    )
    def gather(table_hbm, idx_hbm, out_hbm, idx_v, rows_v, sem):
        wid = lax.axis_index("s") * NC + lax.axis_index("c")
        base = wid * b_per_w

        def chunk(c, _):
            off = base + c * CHUNK
            pltpu.sync_copy(idx_hbm.at[pl.ds(off, CHUNK)], idx_v)
            pltpu.async_copy(table_hbm.at[idx_v], rows_v, sem).wait()
            pltpu.sync_copy(rows_v, out_hbm.at[pl.ds(off, CHUNK)])
            return 0

        jax.lax.fori_loop(0, n_chunks, chunk, 0)

    return gather


def kernel(inputs, embeddings):
    x = inputs
    e = embeddings
    et = e.T
    xp = jnp.sum(jnp.power(x, 2), axis=-1, keepdims=True)
    ep = jnp.sum(jnp.power(et, 2), axis=0, keepdims=True)
    ebt = (et * jnp.float32(-2.0)).astype(jnp.bfloat16)
    code = _argmin_call(x, ebt, xp, ep)
    quant = _make_gather()(e, code)
    return (code, quant)
